# Initial kernel scaffold; baseline (speedup 1.0000x reference)
import functools
import math

import jax
import jax.numpy as jnp
from jax import lax
from jax.experimental import pallas as pl
from jax.experimental.pallas import tpu as pltpu
from jax.experimental.pallas import tpu_sc as plsc

_NC, _NS = 2, 16
_NW = _NC * _NS
_CH = 128
_BE = 2048
_BN = 2000
_LOG2PI = math.log(2.0 * math.pi)
_F32 = jnp.float32


def _round_up(x, m):
    return (x + m - 1) // m * m


def _make_gather2(VA, VB, D, BA, BB):
    pwA, pwB = BA // _NW, BB // _NW
    ncA, ncB = pwA // _CH, pwB // _CH
    assert ncA % 2 == 0 and ncB % 2 == 0
    mesh = plsc.VectorSubcoreMesh(core_axis_name="c", subcore_axis_name="s")

    @functools.partial(
        pl.kernel,
        out_type=(jax.ShapeDtypeStruct((BA, D), _F32),
                  jax.ShapeDtypeStruct((BB, D), _F32)),
        mesh=mesh,
        scratch_types=(
            pltpu.VMEM((ncA, _CH), jnp.int32),
            pltpu.VMEM((ncB, _CH), jnp.int32),
            pltpu.VMEM((_CH, D), _F32),
            pltpu.VMEM((_CH, D), _F32),
            pltpu.SemaphoreType.DMA,
            pltpu.SemaphoreType.DMA,
        ),
    )
    def k(tab_a, idx_a, tab_b, idx_b, out_a, out_b, ixa, ixb, r0, r1, s0, s1):
        wid = lax.axis_index("s") * _NC + lax.axis_index("c")

        def one(tab, idx_hbm, out, ixv, nc, pw):
            base = wid * pw
            pltpu.sync_copy(idx_hbm.at[pl.ds(wid * nc, nc)], ixv)

            @pl.loop(0, nc, step=2)
            def _(j):
                d0 = pltpu.make_async_copy(tab.at[ixv.at[j]], r0, s0)
                d0.start()
                d1 = pltpu.make_async_copy(tab.at[ixv.at[j + 1]], r1, s1)
                d1.start()
                d0.wait()
                pltpu.sync_copy(r0, out.at[pl.ds(base + j * _CH, _CH)])
                d1.wait()
                pltpu.sync_copy(r1, out.at[pl.ds(base + (j + 1) * _CH, _CH)])

        one(tab_a, idx_a, out_a, ixa, ncA, pwA)
        one(tab_b, idx_b, out_b, ixb, ncB, pwB)

    return k


def _make_scatter(EP, NP, D):
    pt = EP // _NS
    nc = pt // _CH
    assert nc % 2 == 0
    zr = NP // _NS
    mesh = plsc.VectorSubcoreMesh(core_axis_name="c", subcore_axis_name="s")

    @functools.partial(
        pl.kernel,
        out_type=jax.ShapeDtypeStruct((2, NP, D), _F32),
        mesh=mesh,
        scratch_types=(
            pltpu.VMEM_SHARED((NP, D), _F32),
            pltpu.VMEM((2, nc, _CH), jnp.int32),
            pltpu.VMEM((_CH, D), _F32),
            pltpu.VMEM((_CH, D), _F32),
            pltpu.SemaphoreType.DMA,
            pltpu.SemaphoreType.DMA,
        ),
    )
    def k(vals, sidx, zeros, out, acc, ixv, v0, v1, s0, s1):
        c = lax.axis_index("c")
        sid = lax.axis_index("s")
        pltpu.sync_copy(zeros.at[pl.ds(sid * zr, zr)], acc.at[pl.ds(sid * zr, zr)])
        pltpu.sync_copy(sidx.at[:, pl.ds(sid * nc, nc)], ixv)
        plsc.subcore_barrier()
        base = sid * pt

        @pl.loop(0, nc, step=2)
        def _(j):
            d0 = pltpu.make_async_copy(vals.at[c, pl.ds(base + j * _CH, _CH)], v0, s0)
            d0.start()
            d1 = pltpu.make_async_copy(
                vals.at[c, pl.ds(base + (j + 1) * _CH, _CH)], v1, s1)
            d1.start()
            d0.wait()
            pltpu.sync_copy(v0, acc.at[ixv.at[0, j]], add=True)
            pltpu.sync_copy(v0, acc.at[ixv.at[1, j]], add=True)
            d1.wait()
            pltpu.sync_copy(v1, acc.at[ixv.at[0, j + 1]], add=True)
            pltpu.sync_copy(v1, acc.at[ixv.at[1, j + 1]], add=True)

        plsc.subcore_barrier()
        pltpu.sync_copy(acc.at[pl.ds(sid * zr, zr)], out.at[c, pl.ds(sid * zr, zr)])

    return k


def _prep_ea(nep, w1ea):
    def body(a_ref, b_ref, o_ref):
        o_ref[...] = jnp.dot(a_ref[...], b_ref[...],
                             preferred_element_type=_F32)
    return pl.pallas_call(
        body,
        out_shape=jax.ShapeDtypeStruct((nep.shape[0], w1ea.shape[1]), _F32),
    )(nep, w1ea)


def _node_phase(aggs, na, W2, b2row, W3ab, Nn, H):
    grid = (Nn // _BN,)

    def body(agg_ref, na_ref, w2_ref, b2_ref, w3_ref, ts_ref, td_ref):
        w2 = w2_ref[...]
        hn = jnp.tanh(jnp.dot(agg_ref[0] + na_ref[...], w2,
                              preferred_element_type=_F32) + b2_ref[...])
        dhn = (1.0 - hn * hn) * jnp.dot(agg_ref[1], w2,
                                        preferred_element_type=_F32)
        r = jnp.dot(jnp.concatenate([hn, dhn], axis=0), w3_ref[...],
                    preferred_element_type=_F32)
        ts_ref[...] = jnp.concatenate([r[:_BN, :H], r[_BN:, :H]], axis=1)
        td_ref[...] = jnp.concatenate([r[:_BN, H:], r[_BN:, H:]], axis=1)

    return pl.pallas_call(
        body,
        grid=grid,
        in_specs=[
            pl.BlockSpec((2, _BN, H), lambda i: (0, i, 0)),
            pl.BlockSpec((_BN, H), lambda i: (i, 0)),
            pl.BlockSpec((H, H), lambda i: (0, 0)),
            pl.BlockSpec((1, H), lambda i: (0, 0)),
            pl.BlockSpec((H, 2 * H), lambda i: (0, 0)),
        ],
        out_specs=[pl.BlockSpec((_BN, 2 * H), lambda i: (i, 0))] * 2,
        out_shape=[jax.ShapeDtypeStruct((Nn, 2 * H), _F32)] * 2,
    )(aggs, na, W2, b2row, W3ab)


def _edge_phase(first, last, EP, H, dt):
    grid = (EP // _BE,)
    evec = pl.BlockSpec((_BE,), lambda i: (i,))
    emat = pl.BlockSpec((_BE, H), lambda i: (i, 0))
    egat = pl.BlockSpec((_BE, 2 * H), lambda i: (i, 0))
    cons = lambda shp: pl.BlockSpec(shp, lambda i: tuple(0 for _ in shp))
    vspec = pl.BlockSpec((2, _BE, H), lambda i: (0, i, 0))

    def body(*refs):
        if first:
            (x_ref, eps_ref, g_ref, p_ref, vals_ref) = refs
        elif last:
            (x_ref, eps_ref, lp_ref, g_ref, gs_ref, gd_ref, p_ref, w3c_ref,
             out_ref) = refs
        else:
            (x_ref, eps_ref, lp_ref, g_ref, gs_ref, gd_ref, p_ref, w3c_ref,
             xo_ref, lpo_ref, vals_ref) = refs

        p = p_ref[...]
        w1x = p[0][None, :]
        ct0 = p[1][None, :]
        ct1 = p[2][None, :]
        b3r = p[3][None, :]
        w4r = p[4][None, :]
        b4s = p[5, 0]
        xv = x_ref[...]
        ev = eps_ref[...]
        xc = lax.broadcast_in_dim(xv, (_BE, H), (0,))
        ec = lax.broadcast_in_dim(ev, (_BE, H), (0,))
        g = g_ref[...]
        demdx = ec * w1x

        if not first:
            m = jnp.tanh(xc * w1x + ct0 + g)
            dm = (1.0 - m * m) * demdx
            s2 = jnp.dot(jnp.concatenate([m, dm], axis=0), w3c_ref[...],
                         preferred_element_type=_F32)
            gs = gs_ref[...]
            gd = gd_ref[...]
            s = gs[:, :H] + gd[:, :H] + s2[:_BE] + b3r
            h = jnp.tanh(s)
            ds = gs[:, H:] + gd[:, H:] + s2[_BE:]
            fx = jnp.sum(h * w4r, axis=1) + b4s
            dout = jnp.sum((1.0 - h * h) * ds * w4r, axis=1)
            xv = xv + dt * fx
            lp = lp_ref[...] + dt * (ev * dout)

        if last:
            out_ref[...] = (-0.5 * _LOG2PI) - 0.5 * xv * xv - lp
        else:
            if not first:
                xo_ref[...] = xv
                lpo_ref[...] = lp
                xc = lax.broadcast_in_dim(xv, (_BE, H), (0,))
            m2 = jnp.tanh(xc * w1x + ct1 + g)
            vals_ref[0] = m2
            vals_ref[1] = (1.0 - m2 * m2) * demdx

    if first:
        in_specs = [evec, evec, emat, cons((8, H))]
        out_specs = vspec
        out_shape = jax.ShapeDtypeStruct((2, EP, H), _F32)
    elif last:
        in_specs = [evec, evec, evec, emat, egat, egat, cons((8, H)),
                    cons((H, H))]
        out_specs = evec
        out_shape = jax.ShapeDtypeStruct((EP,), _F32)
    else:
        in_specs = [evec, evec, evec, emat, egat, egat, cons((8, H)),
                    cons((H, H))]
        out_specs = [evec, evec, vspec]
        out_shape = [jax.ShapeDtypeStruct((EP,), _F32),
                     jax.ShapeDtypeStruct((EP,), _F32),
                     jax.ShapeDtypeStruct((2, EP, H), _F32)]

    return pl.pallas_call(body, grid=grid, in_specs=in_specs,
                          out_specs=out_specs, out_shape=out_shape)


def kernel(d, node_type, edge_type, edge_index, node_emb,
           W1, b1, W2, b2, W3, b3, W4, b4, eps):
    E = d.shape[0]
    Nn = node_type.shape[0]
    V, H = node_emb.shape
    n_steps = 2
    dt = 1.0 / n_steps

    EP = _round_up(E, _NW * _CH * 2)
    NB = _round_up(Nn, _NW * _CH * 2)
    NP = _round_up(Nn + 1, _NS)
    VP = _round_up(V, 8)

    i32 = jnp.int32
    src = edge_index[0].astype(i32)
    dst = edge_index[1].astype(i32)
    epad = EP - E
    src_g = jnp.pad(src, (0, epad)).reshape(EP // _CH, _CH)
    dst_g = jnp.pad(dst, (0, epad)).reshape(EP // _CH, _CH)
    sidx = jnp.stack([
        jnp.pad(src, (0, epad), constant_values=Nn),
        jnp.pad(dst, (0, epad), constant_values=Nn),
    ]).reshape(2, EP // _CH, _CH)
    et2 = jnp.pad(edge_type.astype(i32), (0, epad)).reshape(EP // _CH, _CH)
    nt2 = jnp.pad(node_type.astype(i32), (0, NB - Nn)).reshape(NB // _CH, _CH)

    x = jnp.pad(d[:, 0], (0, epad))
    ev = jnp.pad(eps[:, 0], (0, epad))
    lp = jnp.zeros((EP,), _F32)
    zerosN = jnp.zeros((NP, H), _F32)

    nep = jnp.pad(node_emb, ((0, VP - V), (0, 0)))
    EA = _prep_ea(nep, W1[2:])
    w1x, w1t = W1[0], W1[1]
    w4r = W4[:, 0]
    b4b = jnp.full((H,), b4[0], _F32)
    zrow = jnp.zeros((H,), _F32)
    b2row = b2.reshape(1, H)
    W3ab = jnp.concatenate([W3[:H], W3[H:2 * H]], axis=1)
    W3c = W3[2 * H:]

    def pv(k0, k1):
        return jnp.stack([w1x, (k0 * dt) * w1t + b1, (k1 * dt) * w1t + b1,
                          b3, w4r, b4b, zrow, zrow])

    gather_prep = _make_gather2(VP, VP, H, EP, NB)
    scatter = _make_scatter(EP, NP, H)
    gather_step = _make_gather2(Nn, Nn, 2 * H, EP, EP)

    G, NA = gather_prep(EA, et2, nep, nt2)
    vals = _edge_phase(True, False, EP, H, dt)(x, ev, G, pv(0, 0))

    for k in range(n_steps):
        aggs = scatter(vals, sidx, zerosN)
        Ts, Td = _node_phase(aggs, NA, W2, b2row, W3ab, Nn, H)
        Gs, Gd = gather_step(Ts, src_g, Td, dst_g)
        if k < n_steps - 1:
            x, lp, vals = _edge_phase(False, False, EP, H, dt)(
                x, ev, lp, G, Gs, Gd, pv(k, k + 1), W3c)
        else:
            out = _edge_phase(False, True, EP, H, dt)(
                x, ev, lp, G, Gs, Gd, pv(k, k), W3c)

    return out[:E].reshape(E, 1)


# speedup vs baseline: 1.8287x; 1.8287x over previous
import functools
import math

import jax
import jax.numpy as jnp
from jax import lax
from jax.experimental import pallas as pl
from jax.experimental.pallas import tpu as pltpu
from jax.experimental.pallas import tpu_sc as plsc

_NC, _NS = 2, 16
_NW = _NC * _NS
_CH = 128
_BE = 2048
_BN = 2000
_LOG2PI = math.log(2.0 * math.pi)
_F32 = jnp.float32


def _round_up(x, m):
    return (x + m - 1) // m * m


def _make_gather2(VA, VB, D, BA, BB):
    pwA, pwB = BA // _NW, BB // _NW
    ncA, ncB = pwA // _CH, pwB // _CH
    assert ncA % 2 == 0 and ncB % 2 == 0
    mesh = plsc.VectorSubcoreMesh(core_axis_name="c", subcore_axis_name="s")

    @functools.partial(
        pl.kernel,
        out_type=(jax.ShapeDtypeStruct((BA, D), _F32),
                  jax.ShapeDtypeStruct((BB, D), _F32)),
        mesh=mesh,
        scratch_types=(
            pltpu.VMEM((ncA, _CH), jnp.int32),
            pltpu.VMEM((ncB, _CH), jnp.int32),
            pltpu.VMEM((_CH, D), _F32),
            pltpu.VMEM((_CH, D), _F32),
            pltpu.SemaphoreType.DMA,
            pltpu.SemaphoreType.DMA,
        ),
    )
    def k(tab_a, idx_a, tab_b, idx_b, out_a, out_b, ixa, ixb, r0, r1, s0, s1):
        wid = lax.axis_index("s") * _NC + lax.axis_index("c")

        def one(tab, idx_hbm, out, ixv, nc, pw):
            base = wid * pw
            pltpu.sync_copy(idx_hbm.at[wid], ixv)

            @pl.loop(0, nc, step=2)
            def _(j):
                d0 = pltpu.make_async_copy(tab.at[ixv.at[j]], r0, s0)
                d0.start()
                d1 = pltpu.make_async_copy(tab.at[ixv.at[j + 1]], r1, s1)
                d1.start()
                d0.wait()
                pltpu.sync_copy(r0, out.at[pl.ds(base + j * _CH, _CH)])
                d1.wait()
                pltpu.sync_copy(r1, out.at[pl.ds(base + (j + 1) * _CH, _CH)])

        one(tab_a, idx_a, out_a, ixa, ncA, pwA)
        one(tab_b, idx_b, out_b, ixb, ncB, pwB)

    return k


def _make_scatter(EP, NP, D):
    pt = EP // _NS
    nc = pt // _CH
    assert nc % 8 == 0
    zr = NP // _NS
    mesh = plsc.VectorSubcoreMesh(core_axis_name="c", subcore_axis_name="s")

    @functools.partial(
        pl.kernel,
        out_type=jax.ShapeDtypeStruct((2, NP, D), _F32),
        mesh=mesh,
        scratch_types=(
            pltpu.VMEM_SHARED((NP, D), _F32),
            pltpu.VMEM((2, 8, _CH), jnp.int32),
            pltpu.VMEM((_CH, D), _F32),
            pltpu.VMEM((_CH, D), _F32),
            pltpu.SemaphoreType.DMA,
            pltpu.SemaphoreType.DMA,
        ),
    )
    def k(vals, sidx, zeros, out, acc, ixv, v0, v1, s0, s1):
        c = lax.axis_index("c")
        sid = lax.axis_index("s")
        pltpu.sync_copy(zeros.at[pl.ds(sid * zr, zr)], acc.at[pl.ds(sid * zr, zr)])
        plsc.subcore_barrier()
        base = sid * pt

        @pl.loop(0, nc, step=8)
        def _(j0):
            pltpu.sync_copy(sidx.at[:, sid, pl.ds(j0, 8)], ixv)
            for b in range(0, 8, 2):
                j = j0 + b
                d0 = pltpu.make_async_copy(
                    vals.at[c, pl.ds(base + j * _CH, _CH)], v0, s0)
                d0.start()
                d1 = pltpu.make_async_copy(
                    vals.at[c, pl.ds(base + (j + 1) * _CH, _CH)], v1, s1)
                d1.start()
                d0.wait()
                pltpu.sync_copy(v0, acc.at[ixv.at[0, b]], add=True)
                pltpu.sync_copy(v0, acc.at[ixv.at[1, b]], add=True)
                d1.wait()
                pltpu.sync_copy(v1, acc.at[ixv.at[0, b + 1]], add=True)
                pltpu.sync_copy(v1, acc.at[ixv.at[1, b + 1]], add=True)

        plsc.subcore_barrier()
        pltpu.sync_copy(acc.at[pl.ds(sid * zr, zr)], out.at[c, pl.ds(sid * zr, zr)])

    return k


def _prep_ea(nep, w1ea):
    def body(a_ref, b_ref, o_ref):
        o_ref[...] = jnp.dot(a_ref[...], b_ref[...],
                             preferred_element_type=_F32)
    return pl.pallas_call(
        body,
        out_shape=jax.ShapeDtypeStruct((nep.shape[0], w1ea.shape[1]), _F32),
    )(nep, w1ea)


def _node_phase(aggs, na, W2, b2row, W3ab, Nn, H):
    grid = (Nn // _BN,)

    def body(agg_ref, na_ref, w2_ref, b2_ref, w3_ref, ts_ref, td_ref):
        w2 = w2_ref[...]
        hn = jnp.tanh(jnp.dot(agg_ref[0] + na_ref[...], w2,
                              preferred_element_type=_F32) + b2_ref[...])
        dhn = (1.0 - hn * hn) * jnp.dot(agg_ref[1], w2,
                                        preferred_element_type=_F32)
        r = jnp.dot(jnp.concatenate([hn, dhn], axis=0), w3_ref[...],
                    preferred_element_type=_F32)
        ts_ref[...] = jnp.concatenate([r[:_BN, :H], r[_BN:, :H]], axis=1)
        td_ref[...] = jnp.concatenate([r[:_BN, H:], r[_BN:, H:]], axis=1)

    return pl.pallas_call(
        body,
        grid=grid,
        in_specs=[
            pl.BlockSpec((2, _BN, H), lambda i: (0, i, 0)),
            pl.BlockSpec((_BN, H), lambda i: (i, 0)),
            pl.BlockSpec((H, H), lambda i: (0, 0)),
            pl.BlockSpec((1, H), lambda i: (0, 0)),
            pl.BlockSpec((H, 2 * H), lambda i: (0, 0)),
        ],
        out_specs=[pl.BlockSpec((_BN, 2 * H), lambda i: (i, 0))] * 2,
        out_shape=[jax.ShapeDtypeStruct((Nn, 2 * H), _F32)] * 2,
    )(aggs, na, W2, b2row, W3ab)


def _edge_phase(first, last, EP, H, dt):
    grid = (EP // _BE,)
    evec = pl.BlockSpec((_BE,), lambda i: (i,))
    emat = pl.BlockSpec((_BE, H), lambda i: (i, 0))
    egat = pl.BlockSpec((_BE, 2 * H), lambda i: (i, 0))
    cons = lambda shp: pl.BlockSpec(shp, lambda i: tuple(0 for _ in shp))
    vspec = pl.BlockSpec((2, _BE, H), lambda i: (0, i, 0))

    def body(*refs):
        if first:
            (x_ref, eps_ref, g_ref, p_ref, vals_ref) = refs
        elif last:
            (x_ref, eps_ref, lp_ref, g_ref, gs_ref, gd_ref, p_ref, w3c_ref,
             out_ref) = refs
        else:
            (x_ref, eps_ref, lp_ref, g_ref, gs_ref, gd_ref, p_ref, w3c_ref,
             xo_ref, lpo_ref, vals_ref) = refs

        p = p_ref[...]
        w1x = p[0][None, :]
        ct0 = p[1][None, :]
        ct1 = p[2][None, :]
        b3r = p[3][None, :]
        w4r = p[4][None, :]
        b4s = p[5, 0]
        xv = x_ref[...]
        ev = eps_ref[...]
        xc = lax.broadcast_in_dim(xv, (_BE, H), (0,))
        ec = lax.broadcast_in_dim(ev, (_BE, H), (0,))
        g = g_ref[...]
        demdx = ec * w1x

        if not first:
            m = jnp.tanh(xc * w1x + ct0 + g)
            dm = (1.0 - m * m) * demdx
            s2 = jnp.dot(jnp.concatenate([m, dm], axis=0), w3c_ref[...],
                         preferred_element_type=_F32)
            gs = gs_ref[...]
            gd = gd_ref[...]
            s = gs[:, :H] + gd[:, :H] + s2[:_BE] + b3r
            h = jnp.tanh(s)
            ds = gs[:, H:] + gd[:, H:] + s2[_BE:]
            fx = jnp.sum(h * w4r, axis=1) + b4s
            dout = jnp.sum((1.0 - h * h) * ds * w4r, axis=1)
            xv = xv + dt * fx
            lp = lp_ref[...] + dt * (ev * dout)

        if last:
            out_ref[...] = (-0.5 * _LOG2PI) - 0.5 * xv * xv - lp
        else:
            if not first:
                xo_ref[...] = xv
                lpo_ref[...] = lp
                xc = lax.broadcast_in_dim(xv, (_BE, H), (0,))
            m2 = jnp.tanh(xc * w1x + ct1 + g)
            vals_ref[0] = m2
            vals_ref[1] = (1.0 - m2 * m2) * demdx

    if first:
        in_specs = [evec, evec, emat, cons((8, H))]
        out_specs = vspec
        out_shape = jax.ShapeDtypeStruct((2, EP, H), _F32)
    elif last:
        in_specs = [evec, evec, evec, emat, egat, egat, cons((8, H)),
                    cons((H, H))]
        out_specs = evec
        out_shape = jax.ShapeDtypeStruct((EP,), _F32)
    else:
        in_specs = [evec, evec, evec, emat, egat, egat, cons((8, H)),
                    cons((H, H))]
        out_specs = [evec, evec, vspec]
        out_shape = [jax.ShapeDtypeStruct((EP,), _F32),
                     jax.ShapeDtypeStruct((EP,), _F32),
                     jax.ShapeDtypeStruct((2, EP, H), _F32)]

    return pl.pallas_call(body, grid=grid, in_specs=in_specs,
                          out_specs=out_specs, out_shape=out_shape)


def kernel(d, node_type, edge_type, edge_index, node_emb,
           W1, b1, W2, b2, W3, b3, W4, b4, eps):
    E = d.shape[0]
    Nn = node_type.shape[0]
    V, H = node_emb.shape
    n_steps = 2
    dt = 1.0 / n_steps

    EP = _round_up(E, _NW * _CH * 2)
    NB = _round_up(Nn, _NW * _CH * 2)
    NP = _round_up(Nn + 1, 128)
    VP = _round_up(V, 8)

    i32 = jnp.int32
    src = edge_index[0].astype(i32)
    dst = edge_index[1].astype(i32)
    epad = EP - E
    src_g = jnp.pad(src, (0, epad)).reshape(_NW, -1, _CH)
    dst_g = jnp.pad(dst, (0, epad)).reshape(_NW, -1, _CH)
    sidx = jnp.stack([
        jnp.pad(src, (0, epad), constant_values=Nn),
        jnp.pad(dst, (0, epad), constant_values=Nn),
    ]).reshape(2, _NS, -1, _CH)
    et2 = jnp.pad(edge_type.astype(i32), (0, epad)).reshape(_NW, -1, _CH)
    nt2 = jnp.pad(node_type.astype(i32), (0, NB - Nn)).reshape(_NW, -1, _CH)

    x = jnp.pad(d[:, 0], (0, epad))
    ev = jnp.pad(eps[:, 0], (0, epad))
    lp = jnp.zeros((EP,), _F32)
    zerosN = jnp.zeros((NP, H), _F32)

    nep = jnp.pad(node_emb, ((0, VP - V), (0, 0)))
    EA = _prep_ea(nep, W1[2:])
    w1x, w1t = W1[0], W1[1]
    w4r = W4[:, 0]
    b4b = jnp.full((H,), b4[0], _F32)
    zrow = jnp.zeros((H,), _F32)
    b2row = b2.reshape(1, H)
    W3ab = jnp.concatenate([W3[:H], W3[H:2 * H]], axis=1)
    W3c = W3[2 * H:]

    def pv(k0, k1):
        return jnp.stack([w1x, (k0 * dt) * w1t + b1, (k1 * dt) * w1t + b1,
                          b3, w4r, b4b, zrow, zrow])

    gather_prep = _make_gather2(VP, VP, H, EP, NB)
    scatter = _make_scatter(EP, NP, H)
    gather_step = _make_gather2(Nn, Nn, 2 * H, EP, EP)

    G, NA = gather_prep(EA, et2, nep, nt2)
    vals = _edge_phase(True, False, EP, H, dt)(x, ev, G, pv(0, 0))

    for k in range(n_steps):
        aggs = scatter(vals, sidx, zerosN)
        Ts, Td = _node_phase(aggs, NA, W2, b2row, W3ab, Nn, H)
        Gs, Gd = gather_step(Ts, src_g, Td, dst_g)
        if k < n_steps - 1:
            x, lp, vals = _edge_phase(False, False, EP, H, dt)(
                x, ev, lp, G, Gs, Gd, pv(k, k + 1), W3c)
        else:
            out = _edge_phase(False, True, EP, H, dt)(
                x, ev, lp, G, Gs, Gd, pv(k, k), W3c)

    return out[:E].reshape(E, 1)


# speedup vs baseline: 1.9323x; 1.0567x over previous
import functools
import math

import jax
import jax.numpy as jnp
from jax import lax
from jax.experimental import pallas as pl
from jax.experimental.pallas import tpu as pltpu
from jax.experimental.pallas import tpu_sc as plsc

_NC, _NS = 2, 16
_NW = _NC * _NS
_CH = 128
_BE = 2048
_BN = 2000
_LOG2PI = math.log(2.0 * math.pi)
_F32 = jnp.float32


def _round_up(x, m):
    return (x + m - 1) // m * m


def _make_gather2(VA, VB, D, BA, BB):
    pwA, pwB = BA // _NW, BB // _NW
    ncA, ncB = pwA // _CH, pwB // _CH
    assert ncA % 2 == 0 and ncB % 2 == 0
    mesh = plsc.VectorSubcoreMesh(core_axis_name="c", subcore_axis_name="s")

    @functools.partial(
        pl.kernel,
        out_type=(jax.ShapeDtypeStruct((BA, D), _F32),
                  jax.ShapeDtypeStruct((BB, D), _F32)),
        mesh=mesh,
        scratch_types=(
            pltpu.VMEM((ncA, _CH), jnp.int32),
            pltpu.VMEM((ncB, _CH), jnp.int32),
            pltpu.VMEM((_CH, D), _F32),
            pltpu.VMEM((_CH, D), _F32),
            pltpu.SemaphoreType.DMA,
            pltpu.SemaphoreType.DMA,
            pltpu.SemaphoreType.DMA,
            pltpu.SemaphoreType.DMA,
        ),
    )
    def k(tab_a, idx_a, tab_b, idx_b, out_a, out_b, ixa, ixb, r0, r1,
          s0, s1, t0, t1):
        wid = lax.axis_index("s") * _NC + lax.axis_index("c")

        def one(tab, idx_hbm, out, ixv, nc, pw):
            base = wid * pw
            pltpu.sync_copy(idx_hbm.at[wid], ixv)

            def gat(j, buf, sem):
                return pltpu.make_async_copy(tab.at[ixv.at[j]], buf, sem)

            def sto(j, buf, sem):
                return pltpu.make_async_copy(
                    buf, out.at[pl.ds(base + j * _CH, _CH)], sem)

            gat(0, r0, s0).start()
            gat(1, r1, s1).start()

            @pl.loop(0, nc, step=2)
            def _(j):
                gat(j, r0, s0).wait()
                sto(j, r0, t0).start()
                gat(j + 1, r1, s1).wait()
                sto(j + 1, r1, t1).start()
                sto(j, r0, t0).wait()

                @pl.when(j + 2 < nc)
                def _():
                    gat(j + 2, r0, s0).start()
                sto(j + 1, r1, t1).wait()

                @pl.when(j + 3 < nc)
                def _():
                    gat(j + 3, r1, s1).start()

        one(tab_a, idx_a, out_a, ixa, ncA, pwA)
        one(tab_b, idx_b, out_b, ixb, ncB, pwB)

    return k


def _make_scatter(EP, NP, D):
    pt = EP // _NS
    nc = pt // _CH
    assert nc % 8 == 0
    zr = NP // _NS
    nch = nc // 2
    mesh = plsc.VectorSubcoreMesh(core_axis_name="c", subcore_axis_name="s")

    @functools.partial(
        pl.kernel,
        out_type=jax.ShapeDtypeStruct((2, NP, D), _F32),
        mesh=mesh,
        scratch_types=(
            pltpu.VMEM_SHARED((NP, D), _F32),
            pltpu.VMEM((2, nc // 2, _CH), jnp.int32),
            pltpu.VMEM((_CH, D), _F32),
            pltpu.VMEM((_CH, D), _F32),
            pltpu.SemaphoreType.DMA,
            pltpu.SemaphoreType.DMA,
            pltpu.SemaphoreType.DMA,
            pltpu.SemaphoreType.DMA,
        ),
    )
    def k(vals, sidx, zeros, out, acc, ixv, v0, v1, s0, s1, a0, a1):
        c = lax.axis_index("c")
        sid = lax.axis_index("s")
        pltpu.sync_copy(zeros.at[pl.ds(sid * zr, zr)], acc.at[pl.ds(sid * zr, zr)])
        plsc.subcore_barrier()
        base = sid * pt

        def lod(g, buf, sem):
            return pltpu.make_async_copy(
                vals.at[c, pl.ds(base + g * _CH, _CH)], buf, sem)

        def add(i, jloc, buf, sem):
            return pltpu.make_async_copy(buf, acc.at[ixv.at[i, jloc]], sem)

        lod(0, v0, s0).start()
        lod(1, v1, s1).start()

        for h in range(2):
            pltpu.sync_copy(sidx.at[:, sid, pl.ds(h * nch, nch)], ixv)

            @pl.loop(0, nch, step=2)
            def _(j):
                g = h * nch + j
                lod(g, v0, s0).wait()
                add(0, j, v0, a0).start(add=True)
                add(1, j, v0, a0).start(add=True)
                lod(g + 1, v1, s1).wait()
                add(0, j + 1, v1, a1).start(add=True)
                add(1, j + 1, v1, a1).start(add=True)
                add(0, j, v0, a0).wait()
                add(1, j, v0, a0).wait()

                @pl.when(g + 2 < nc)
                def _():
                    lod(g + 2, v0, s0).start()
                add(0, j + 1, v1, a1).wait()
                add(1, j + 1, v1, a1).wait()

                @pl.when(g + 3 < nc)
                def _():
                    lod(g + 3, v1, s1).start()

        plsc.subcore_barrier()
        pltpu.sync_copy(acc.at[pl.ds(sid * zr, zr)], out.at[c, pl.ds(sid * zr, zr)])

    return k


def _prep_ea(nep, w1ea):
    def body(a_ref, b_ref, o_ref):
        o_ref[...] = jnp.dot(a_ref[...], b_ref[...],
                             preferred_element_type=_F32)
    return pl.pallas_call(
        body,
        out_shape=jax.ShapeDtypeStruct((nep.shape[0], w1ea.shape[1]), _F32),
    )(nep, w1ea)


def _node_phase(aggs, na, W2, b2row, W3ab, Nn, H):
    grid = (Nn // _BN,)

    def body(agg_ref, na_ref, w2_ref, b2_ref, w3_ref, ts_ref, td_ref):
        w2 = w2_ref[...]
        hn = jnp.tanh(jnp.dot(agg_ref[0] + na_ref[...], w2,
                              preferred_element_type=_F32) + b2_ref[...])
        dhn = (1.0 - hn * hn) * jnp.dot(agg_ref[1], w2,
                                        preferred_element_type=_F32)
        r = jnp.dot(jnp.concatenate([hn, dhn], axis=0), w3_ref[...],
                    preferred_element_type=_F32)
        ts_ref[...] = jnp.concatenate([r[:_BN, :H], r[_BN:, :H]], axis=1)
        td_ref[...] = jnp.concatenate([r[:_BN, H:], r[_BN:, H:]], axis=1)

    return pl.pallas_call(
        body,
        grid=grid,
        in_specs=[
            pl.BlockSpec((2, _BN, H), lambda i: (0, i, 0)),
            pl.BlockSpec((_BN, H), lambda i: (i, 0)),
            pl.BlockSpec((H, H), lambda i: (0, 0)),
            pl.BlockSpec((1, H), lambda i: (0, 0)),
            pl.BlockSpec((H, 2 * H), lambda i: (0, 0)),
        ],
        out_specs=[pl.BlockSpec((_BN, 2 * H), lambda i: (i, 0))] * 2,
        out_shape=[jax.ShapeDtypeStruct((Nn, 2 * H), _F32)] * 2,
    )(aggs, na, W2, b2row, W3ab)


def _edge_phase(first, last, EP, H, dt):
    grid = (EP // _BE,)
    evec = pl.BlockSpec((_BE,), lambda i: (i,))
    emat = pl.BlockSpec((_BE, H), lambda i: (i, 0))
    egat = pl.BlockSpec((_BE, 2 * H), lambda i: (i, 0))
    cons = lambda shp: pl.BlockSpec(shp, lambda i: tuple(0 for _ in shp))
    vspec = pl.BlockSpec((2, _BE, H), lambda i: (0, i, 0))

    def body(*refs):
        if first:
            (x_ref, eps_ref, g_ref, p_ref, vals_ref) = refs
        elif last:
            (x_ref, eps_ref, lp_ref, g_ref, gs_ref, gd_ref, p_ref, w3c_ref,
             out_ref) = refs
        else:
            (x_ref, eps_ref, lp_ref, g_ref, gs_ref, gd_ref, p_ref, w3c_ref,
             xo_ref, lpo_ref, vals_ref) = refs

        p = p_ref[...]
        w1x = p[0][None, :]
        ct0 = p[1][None, :]
        ct1 = p[2][None, :]
        b3r = p[3][None, :]
        w4r = p[4][None, :]
        b4s = p[5, 0]
        xc = x_ref[...][:, None]
        ec = eps_ref[...][:, None]
        g = g_ref[...]
        demdx = ec * w1x

        if not first:
            m = jnp.tanh(xc * w1x + ct0 + g)
            dm = (1.0 - m * m) * demdx
            s2 = jnp.dot(jnp.concatenate([m, dm], axis=0), w3c_ref[...],
                         preferred_element_type=_F32)
            gs = gs_ref[...]
            gd = gd_ref[...]
            s = gs[:, :H] + gd[:, :H] + s2[:_BE] + b3r
            h = jnp.tanh(s)
            ds = gs[:, H:] + gd[:, H:] + s2[_BE:]
            fx = jnp.sum(h * w4r, axis=1, keepdims=True) + b4s
            dout = jnp.sum((1.0 - h * h) * ds * w4r, axis=1, keepdims=True)
            xc = xc + dt * fx
            lp = lp_ref[...][:, None] + dt * (ec * dout)

        if last:
            o = (-0.5 * _LOG2PI) - 0.5 * xc * xc - lp
            out_ref[...] = o[:, 0]
        else:
            if not first:
                xo_ref[...] = xc[:, 0]
                lpo_ref[...] = lp[:, 0]
            m2 = jnp.tanh(xc * w1x + ct1 + g)
            vals_ref[0] = m2
            vals_ref[1] = (1.0 - m2 * m2) * demdx

    if first:
        in_specs = [evec, evec, emat, cons((8, H))]
        out_specs = vspec
        out_shape = jax.ShapeDtypeStruct((2, EP, H), _F32)
    elif last:
        in_specs = [evec, evec, evec, emat, egat, egat, cons((8, H)),
                    cons((H, H))]
        out_specs = evec
        out_shape = jax.ShapeDtypeStruct((EP,), _F32)
    else:
        in_specs = [evec, evec, evec, emat, egat, egat, cons((8, H)),
                    cons((H, H))]
        out_specs = [evec, evec, vspec]
        out_shape = [jax.ShapeDtypeStruct((EP,), _F32),
                     jax.ShapeDtypeStruct((EP,), _F32),
                     jax.ShapeDtypeStruct((2, EP, H), _F32)]

    return pl.pallas_call(body, grid=grid, in_specs=in_specs,
                          out_specs=out_specs, out_shape=out_shape)


def kernel(d, node_type, edge_type, edge_index, node_emb,
           W1, b1, W2, b2, W3, b3, W4, b4, eps):
    E = d.shape[0]
    Nn = node_type.shape[0]
    V, H = node_emb.shape
    n_steps = 2
    dt = 1.0 / n_steps

    EP = _round_up(E, _NW * _CH * 2)
    NB = _round_up(Nn, _NW * _CH * 2)
    NP = _round_up(Nn + 1, 128)
    VP = _round_up(V, 8)

    i32 = jnp.int32
    src = edge_index[0].astype(i32)
    dst = edge_index[1].astype(i32)
    epad = EP - E
    src_g = jnp.pad(src, (0, epad)).reshape(_NW, -1, _CH)
    dst_g = jnp.pad(dst, (0, epad)).reshape(_NW, -1, _CH)
    sidx = jnp.stack([
        jnp.pad(src, (0, epad), constant_values=Nn),
        jnp.pad(dst, (0, epad), constant_values=Nn),
    ]).reshape(2, _NS, -1, _CH)
    et2 = jnp.pad(edge_type.astype(i32), (0, epad)).reshape(_NW, -1, _CH)
    nt2 = jnp.pad(node_type.astype(i32), (0, NB - Nn)).reshape(_NW, -1, _CH)

    x = jnp.pad(d[:, 0], (0, epad))
    ev = jnp.pad(eps[:, 0], (0, epad))
    lp = jnp.zeros((EP,), _F32)
    zerosN = jnp.zeros((NP, H), _F32)

    nep = jnp.pad(node_emb, ((0, VP - V), (0, 0)))
    EA = _prep_ea(nep, W1[2:])
    w1x, w1t = W1[0], W1[1]
    w4r = W4[:, 0]
    b4b = jnp.full((H,), b4[0], _F32)
    zrow = jnp.zeros((H,), _F32)
    b2row = b2.reshape(1, H)
    W3ab = jnp.concatenate([W3[:H], W3[H:2 * H]], axis=1)
    W3c = W3[2 * H:]

    def pv(k0, k1):
        return jnp.stack([w1x, (k0 * dt) * w1t + b1, (k1 * dt) * w1t + b1,
                          b3, w4r, b4b, zrow, zrow])

    gather_prep = _make_gather2(VP, VP, H, EP, NB)
    scatter = _make_scatter(EP, NP, H)
    gather_step = _make_gather2(Nn, Nn, 2 * H, EP, EP)

    G, NA = gather_prep(EA, et2, nep, nt2)
    vals = _edge_phase(True, False, EP, H, dt)(x, ev, G, pv(0, 0))

    for k in range(n_steps):
        aggs = scatter(vals, sidx, zerosN)
        Ts, Td = _node_phase(aggs, NA, W2, b2row, W3ab, Nn, H)
        Gs, Gd = gather_step(Ts, src_g, Td, dst_g)
        if k < n_steps - 1:
            x, lp, vals = _edge_phase(False, False, EP, H, dt)(
                x, ev, lp, G, Gs, Gd, pv(k, k + 1), W3c)
        else:
            out = _edge_phase(False, True, EP, H, dt)(
                x, ev, lp, G, Gs, Gd, pv(k, k), W3c)

    return out[:E].reshape(E, 1)


# speedup vs baseline: 3.5985x; 1.8623x over previous
import functools
import math

import jax
import jax.numpy as jnp
from jax import lax
from jax.experimental import pallas as pl
from jax.experimental.pallas import tpu as pltpu
from jax.experimental.pallas import tpu_sc as plsc

_NC, _NS = 2, 16
_NW = _NC * _NS
_CH = 128
_BE = 2048
_BN = 2000
_LOG2PI = math.log(2.0 * math.pi)
_F32 = jnp.float32


def _round_up(x, m):
    return (x + m - 1) // m * m


def _gather_pipe(gat, sto, nc):
    gat(0, 0).start()
    gat(1, 1).start()

    @pl.loop(0, nc, step=2)
    def _(j):
        gat(j, 0).wait()
        sto(j, 0).start()
        gat(j + 1, 1).wait()
        sto(j + 1, 1).start()
        sto(j, 0).wait()

        @pl.when(j + 2 < nc)
        def _():
            gat(j + 2, 0).start()
        sto(j + 1, 1).wait()

        @pl.when(j + 3 < nc)
        def _():
            gat(j + 3, 1).start()


def _make_gather_small(VA, VB, D, BA, BB):
    pwA, pwB = BA // _NW, BB // _NW
    ncA, ncB = pwA // _CH, pwB // _CH
    assert ncA % 2 == 0 and ncB % 2 == 0
    mesh = plsc.VectorSubcoreMesh(core_axis_name="c", subcore_axis_name="s")

    @functools.partial(
        pl.kernel,
        out_type=(jax.ShapeDtypeStruct((BA, D), _F32),
                  jax.ShapeDtypeStruct((BB, D), _F32)),
        mesh=mesh,
        scratch_types=(
            pltpu.VMEM_SHARED((VA, D), _F32),
            pltpu.VMEM_SHARED((VB, D), _F32),
            pltpu.VMEM((ncA, _CH), jnp.int32),
            pltpu.VMEM((ncB, _CH), jnp.int32),
            pltpu.VMEM((_CH, D), _F32),
            pltpu.VMEM((_CH, D), _F32),
            pltpu.SemaphoreType.DMA,
            pltpu.SemaphoreType.DMA,
            pltpu.SemaphoreType.DMA,
            pltpu.SemaphoreType.DMA,
        ),
    )
    def k(tab_a, idx_a, tab_b, idx_b, out_a, out_b, spa, spb, ixa, ixb,
          r0, r1, s0, s1, t0, t1):
        c = lax.axis_index("c")
        sid = lax.axis_index("s")
        wid = c * _NS + sid

        @pl.when(sid == 0)
        def _():
            pltpu.sync_copy(tab_a, spa)
            pltpu.sync_copy(tab_b, spb)
        plsc.subcore_barrier()

        def one(sp, idx_hbm, out, ixv, nc, pw):
            base = wid * pw
            pltpu.sync_copy(idx_hbm.at[wid], ixv)
            bufs = (r0, r1)
            gsem = (s0, s1)
            ssem = (t0, t1)

            def gat(j, ping):
                return pltpu.make_async_copy(sp.at[ixv.at[j]], bufs[ping],
                                             gsem[ping])

            def sto(j, ping):
                return pltpu.make_async_copy(
                    bufs[ping], out.at[pl.ds(base + j * _CH, _CH)], ssem[ping])

            _gather_pipe(gat, sto, nc)

        one(spa, idx_a, out_a, ixa, ncA, pwA)
        one(spb, idx_b, out_b, ixb, ncB, pwB)

    return k


def _make_gather_big(V, D, B):
    D2 = D // 2
    pw = B // _NW
    nc = pw // _CH
    assert nc % 2 == 0
    srow = _round_up((V + _NS - 1) // _NS, 8)
    mesh = plsc.VectorSubcoreMesh(core_axis_name="c", subcore_axis_name="s")

    @functools.partial(
        pl.kernel,
        out_type=(jax.ShapeDtypeStruct((B, D), _F32),
                  jax.ShapeDtypeStruct((B, D), _F32)),
        mesh=mesh,
        scratch_types=(
            pltpu.VMEM_SHARED((V, D2), _F32),
            pltpu.VMEM((nc, _CH), jnp.int32),
            pltpu.VMEM((_CH, D2), _F32),
            pltpu.VMEM((_CH, D2), _F32),
            pltpu.SemaphoreType.DMA,
            pltpu.SemaphoreType.DMA,
            pltpu.SemaphoreType.DMA,
            pltpu.SemaphoreType.DMA,
        ),
    )
    def k(tab_a, idx_a, tab_b, idx_b, out_a, out_b, sp, ixv, r0, r1,
          s0, s1, t0, t1):
        c = lax.axis_index("c")
        sid = lax.axis_index("s")
        wid = c * _NS + sid
        base = wid * pw
        row0 = jnp.minimum(sid * srow, V - srow)
        bufs = (r0, r1)
        gsem = (s0, s1)
        ssem = (t0, t1)

        for tab, idx_hbm, out in ((tab_a, idx_a, out_a),
                                  (tab_b, idx_b, out_b)):
            pltpu.sync_copy(idx_hbm.at[wid], ixv)
            for half in range(2):
                plsc.subcore_barrier()
                pltpu.sync_copy(
                    tab.at[pl.ds(row0, srow), pl.ds(half * D2, D2)],
                    sp.at[pl.ds(row0, srow)])
                plsc.subcore_barrier()

                def gat(j, ping):
                    return pltpu.make_async_copy(sp.at[ixv.at[j]],
                                                 bufs[ping], gsem[ping])

                def sto(j, ping):
                    return pltpu.make_async_copy(
                        bufs[ping],
                        out.at[pl.ds(base + j * _CH, _CH),
                               pl.ds(half * D2, D2)],
                        ssem[ping])

                _gather_pipe(gat, sto, nc)

    return k


def _make_scatter(EP, NP, D):
    pt = EP // _NS
    nc = pt // _CH
    assert nc % 8 == 0
    zr = NP // _NS
    nch = nc // 2
    mesh = plsc.VectorSubcoreMesh(core_axis_name="c", subcore_axis_name="s")

    @functools.partial(
        pl.kernel,
        out_type=jax.ShapeDtypeStruct((2, NP, D), _F32),
        mesh=mesh,
        scratch_types=(
            pltpu.VMEM_SHARED((NP, D), _F32),
            pltpu.VMEM((2, nc // 2, _CH), jnp.int32),
            pltpu.VMEM((_CH, D), _F32),
            pltpu.VMEM((_CH, D), _F32),
            pltpu.SemaphoreType.DMA,
            pltpu.SemaphoreType.DMA,
            pltpu.SemaphoreType.DMA,
            pltpu.SemaphoreType.DMA,
        ),
    )
    def k(vals, sidx, zeros, out, acc, ixv, v0, v1, s0, s1, a0, a1):
        c = lax.axis_index("c")
        sid = lax.axis_index("s")
        pltpu.sync_copy(zeros.at[pl.ds(sid * zr, zr)], acc.at[pl.ds(sid * zr, zr)])
        plsc.subcore_barrier()
        base = sid * pt

        def lod(g, buf, sem):
            return pltpu.make_async_copy(
                vals.at[c, pl.ds(base + g * _CH, _CH)], buf, sem)

        def add(i, jloc, buf, sem):
            return pltpu.make_async_copy(buf, acc.at[ixv.at[i, jloc]], sem)

        lod(0, v0, s0).start()
        lod(1, v1, s1).start()

        for h in range(2):
            pltpu.sync_copy(sidx.at[:, sid, pl.ds(h * nch, nch)], ixv)

            @pl.loop(0, nch, step=2)
            def _(j):
                g = h * nch + j
                lod(g, v0, s0).wait()
                add(0, j, v0, a0).start(add=True)
                add(1, j, v0, a0).start(add=True)
                lod(g + 1, v1, s1).wait()
                add(0, j + 1, v1, a1).start(add=True)
                add(1, j + 1, v1, a1).start(add=True)
                add(0, j, v0, a0).wait()
                add(1, j, v0, a0).wait()

                @pl.when(g + 2 < nc)
                def _():
                    lod(g + 2, v0, s0).start()
                add(0, j + 1, v1, a1).wait()
                add(1, j + 1, v1, a1).wait()

                @pl.when(g + 3 < nc)
                def _():
                    lod(g + 3, v1, s1).start()

        plsc.subcore_barrier()
        pltpu.sync_copy(acc.at[pl.ds(sid * zr, zr)], out.at[c, pl.ds(sid * zr, zr)])

    return k


def _prep_ea(nep, w1ea):
    def body(a_ref, b_ref, o_ref):
        o_ref[...] = jnp.dot(a_ref[...], b_ref[...],
                             preferred_element_type=_F32)
    return pl.pallas_call(
        body,
        out_shape=jax.ShapeDtypeStruct((nep.shape[0], w1ea.shape[1]), _F32),
    )(nep, w1ea)


def _node_phase(aggs, na, W2, b2row, W3ab, Nn, H):
    grid = (Nn // _BN,)

    def body(agg_ref, na_ref, w2_ref, b2_ref, w3_ref, ts_ref, td_ref):
        w2 = w2_ref[...]
        hn = jnp.tanh(jnp.dot(agg_ref[0] + na_ref[...], w2,
                              preferred_element_type=_F32) + b2_ref[...])
        dhn = (1.0 - hn * hn) * jnp.dot(agg_ref[1], w2,
                                        preferred_element_type=_F32)
        r = jnp.dot(jnp.concatenate([hn, dhn], axis=0), w3_ref[...],
                    preferred_element_type=_F32)
        ts_ref[...] = jnp.concatenate([r[:_BN, :H], r[_BN:, :H]], axis=1)
        td_ref[...] = jnp.concatenate([r[:_BN, H:], r[_BN:, H:]], axis=1)

    return pl.pallas_call(
        body,
        grid=grid,
        in_specs=[
            pl.BlockSpec((2, _BN, H), lambda i: (0, i, 0)),
            pl.BlockSpec((_BN, H), lambda i: (i, 0)),
            pl.BlockSpec((H, H), lambda i: (0, 0)),
            pl.BlockSpec((1, H), lambda i: (0, 0)),
            pl.BlockSpec((H, 2 * H), lambda i: (0, 0)),
        ],
        out_specs=[pl.BlockSpec((_BN, 2 * H), lambda i: (i, 0))] * 2,
        out_shape=[jax.ShapeDtypeStruct((Nn, 2 * H), _F32)] * 2,
    )(aggs, na, W2, b2row, W3ab)


def _edge_phase(first, last, EP, H, dt):
    grid = (EP // _BE,)
    evec = pl.BlockSpec((_BE,), lambda i: (i,))
    emat = pl.BlockSpec((_BE, H), lambda i: (i, 0))
    egat = pl.BlockSpec((_BE, 2 * H), lambda i: (i, 0))
    cons = lambda shp: pl.BlockSpec(shp, lambda i: tuple(0 for _ in shp))
    vspec = pl.BlockSpec((2, _BE, H), lambda i: (0, i, 0))

    def body(*refs):
        if first:
            (x_ref, eps_ref, g_ref, p_ref, vals_ref) = refs
        elif last:
            (x_ref, eps_ref, lp_ref, g_ref, gs_ref, gd_ref, p_ref, w3c_ref,
             out_ref) = refs
        else:
            (x_ref, eps_ref, lp_ref, g_ref, gs_ref, gd_ref, p_ref, w3c_ref,
             xo_ref, lpo_ref, vals_ref) = refs

        p = p_ref[...]
        w1x = p[0][None, :]
        ct0 = p[1][None, :]
        ct1 = p[2][None, :]
        b3r = p[3][None, :]
        w4r = p[4][None, :]
        b4s = p[5, 0]
        xc = x_ref[...][:, None]
        ec = eps_ref[...][:, None]
        g = g_ref[...]
        demdx = ec * w1x

        if not first:
            m = jnp.tanh(xc * w1x + ct0 + g)
            dm = (1.0 - m * m) * demdx
            s2 = jnp.dot(jnp.concatenate([m, dm], axis=0), w3c_ref[...],
                         preferred_element_type=_F32)
            gs = gs_ref[...]
            gd = gd_ref[...]
            s = gs[:, :H] + gd[:, :H] + s2[:_BE] + b3r
            h = jnp.tanh(s)
            ds = gs[:, H:] + gd[:, H:] + s2[_BE:]
            fx = jnp.sum(h * w4r, axis=1, keepdims=True) + b4s
            dout = jnp.sum((1.0 - h * h) * ds * w4r, axis=1, keepdims=True)
            xc = xc + dt * fx
            lp = lp_ref[...][:, None] + dt * (ec * dout)

        if last:
            o = (-0.5 * _LOG2PI) - 0.5 * xc * xc - lp
            out_ref[...] = o[:, 0]
        else:
            if not first:
                xo_ref[...] = xc[:, 0]
                lpo_ref[...] = lp[:, 0]
            m2 = jnp.tanh(xc * w1x + ct1 + g)
            vals_ref[0] = m2
            vals_ref[1] = (1.0 - m2 * m2) * demdx

    if first:
        in_specs = [evec, evec, emat, cons((8, H))]
        out_specs = vspec
        out_shape = jax.ShapeDtypeStruct((2, EP, H), _F32)
    elif last:
        in_specs = [evec, evec, evec, emat, egat, egat, cons((8, H)),
                    cons((H, H))]
        out_specs = evec
        out_shape = jax.ShapeDtypeStruct((EP,), _F32)
    else:
        in_specs = [evec, evec, evec, emat, egat, egat, cons((8, H)),
                    cons((H, H))]
        out_specs = [evec, evec, vspec]
        out_shape = [jax.ShapeDtypeStruct((EP,), _F32),
                     jax.ShapeDtypeStruct((EP,), _F32),
                     jax.ShapeDtypeStruct((2, EP, H), _F32)]

    return pl.pallas_call(body, grid=grid, in_specs=in_specs,
                          out_specs=out_specs, out_shape=out_shape)


def kernel(d, node_type, edge_type, edge_index, node_emb,
           W1, b1, W2, b2, W3, b3, W4, b4, eps):
    E = d.shape[0]
    Nn = node_type.shape[0]
    V, H = node_emb.shape
    n_steps = 2
    dt = 1.0 / n_steps

    EP = _round_up(E, _NW * _CH * 2)
    NB = _round_up(Nn, _NW * _CH * 2)
    NP = _round_up(Nn + 1, 128)
    VP = _round_up(V, 8)

    i32 = jnp.int32
    src = edge_index[0].astype(i32)
    dst = edge_index[1].astype(i32)
    epad = EP - E
    src_g = jnp.pad(src, (0, epad)).reshape(_NW, -1, _CH)
    dst_g = jnp.pad(dst, (0, epad)).reshape(_NW, -1, _CH)
    sidx = jnp.stack([
        jnp.pad(src, (0, epad), constant_values=Nn),
        jnp.pad(dst, (0, epad), constant_values=Nn),
    ]).reshape(2, _NS, -1, _CH)
    et2 = jnp.pad(edge_type.astype(i32), (0, epad)).reshape(_NW, -1, _CH)
    nt2 = jnp.pad(node_type.astype(i32), (0, NB - Nn)).reshape(_NW, -1, _CH)

    x = jnp.pad(d[:, 0], (0, epad))
    ev = jnp.pad(eps[:, 0], (0, epad))
    lp = jnp.zeros((EP,), _F32)
    zerosN = jnp.zeros((NP, H), _F32)

    nep = jnp.pad(node_emb, ((0, VP - V), (0, 0)))
    EA = _prep_ea(nep, W1[2:])
    w1x, w1t = W1[0], W1[1]
    w4r = W4[:, 0]
    b4b = jnp.full((H,), b4[0], _F32)
    zrow = jnp.zeros((H,), _F32)
    b2row = b2.reshape(1, H)
    W3ab = jnp.concatenate([W3[:H], W3[H:2 * H]], axis=1)
    W3c = W3[2 * H:]

    def pv(k0, k1):
        return jnp.stack([w1x, (k0 * dt) * w1t + b1, (k1 * dt) * w1t + b1,
                          b3, w4r, b4b, zrow, zrow])

    gather_prep = _make_gather_small(VP, VP, H, EP, NB)
    scatter = _make_scatter(EP, NP, H)
    gather_step = _make_gather_big(Nn, 2 * H, EP)

    G, NA = gather_prep(EA, et2, nep, nt2)
    vals = _edge_phase(True, False, EP, H, dt)(x, ev, G, pv(0, 0))

    for k in range(n_steps):
        aggs = scatter(vals, sidx, zerosN)
        Ts, Td = _node_phase(aggs, NA, W2, b2row, W3ab, Nn, H)
        Gs, Gd = gather_step(Ts, src_g, Td, dst_g)
        if k < n_steps - 1:
            x, lp, vals = _edge_phase(False, False, EP, H, dt)(
                x, ev, lp, G, Gs, Gd, pv(k, k + 1), W3c)
        else:
            out = _edge_phase(False, True, EP, H, dt)(
                x, ev, lp, G, Gs, Gd, pv(k, k), W3c)

    return out[:E].reshape(E, 1)


# speedup vs baseline: 4.1912x; 1.1647x over previous
import functools
import math

import jax
import jax.numpy as jnp
from jax import lax
from jax.experimental import pallas as pl
from jax.experimental.pallas import tpu as pltpu
from jax.experimental.pallas import tpu_sc as plsc

_NC, _NS = 2, 16
_NW = _NC * _NS
_CH = 128
_BE = 2048
_BN = 2000
_LOG2PI = math.log(2.0 * math.pi)
_F32 = jnp.float32


def _round_up(x, m):
    return (x + m - 1) // m * m


def _gather_pipe(gat, sto, nc):
    gat(0, 0).start()
    gat(1, 1).start()

    @pl.loop(0, nc, step=2)
    def _(j):
        gat(j, 0).wait()
        sto(j, 0).start()
        gat(j + 1, 1).wait()
        sto(j + 1, 1).start()
        sto(j, 0).wait()

        @pl.when(j + 2 < nc)
        def _():
            gat(j + 2, 0).start()
        sto(j + 1, 1).wait()

        @pl.when(j + 3 < nc)
        def _():
            gat(j + 3, 1).start()


def _make_gather_small(specs, D):
    pws = [B // _NW for _, B in specs]
    ncs = [pw // _CH for pw in pws]
    assert all(nc % 2 == 0 for nc in ncs)
    mesh = plsc.VectorSubcoreMesh(core_axis_name="c", subcore_axis_name="s")

    @functools.partial(
        pl.kernel,
        out_type=tuple(jax.ShapeDtypeStruct((B, D), _F32) for _, B in specs),
        mesh=mesh,
        scratch_types=tuple(
            [pltpu.VMEM_SHARED((V, D), _F32) for V, _ in specs]
            + [pltpu.VMEM((nc, _CH), jnp.int32) for nc in ncs]
            + [pltpu.VMEM((_CH, D), _F32), pltpu.VMEM((_CH, D), _F32),
               pltpu.SemaphoreType.DMA, pltpu.SemaphoreType.DMA,
               pltpu.SemaphoreType.DMA, pltpu.SemaphoreType.DMA]),
    )
    def k(*refs):
        n = len(specs)
        tabs = refs[0:2 * n:2]
        idxs = refs[1:2 * n:2]
        outs = refs[2 * n:3 * n]
        sps = refs[3 * n:4 * n]
        ixvs = refs[4 * n:5 * n]
        r0, r1, s0, s1, t0, t1 = refs[5 * n:]
        c = lax.axis_index("c")
        sid = lax.axis_index("s")
        wid = c * _NS + sid

        @pl.when(sid == 0)
        def _():
            for tab, sp in zip(tabs, sps):
                pltpu.sync_copy(tab, sp)
        plsc.subcore_barrier()
        bufs = (r0, r1)
        gsem = (s0, s1)
        ssem = (t0, t1)

        for i in range(n):
            sp, out, ixv, nc, pw = sps[i], outs[i], ixvs[i], ncs[i], pws[i]
            base = wid * pw
            pltpu.sync_copy(idxs[i].at[wid], ixv)

            def gat(j, ping, sp=sp, ixv=ixv):
                return pltpu.make_async_copy(sp.at[ixv.at[j]], bufs[ping],
                                             gsem[ping])

            def sto(j, ping, out=out, base=base):
                return pltpu.make_async_copy(
                    bufs[ping], out.at[pl.ds(base + j * _CH, _CH)], ssem[ping])

            _gather_pipe(gat, sto, nc)

    return k


def _make_gather_big(V, D, B):
    D2 = D // 2
    pw = B // _NW
    nc = pw // _CH
    assert nc % 2 == 0
    srow = _round_up((V + _NS - 1) // _NS, 8)
    mesh = plsc.VectorSubcoreMesh(core_axis_name="c", subcore_axis_name="s")

    @functools.partial(
        pl.kernel,
        out_type=(jax.ShapeDtypeStruct((B, D), _F32),
                  jax.ShapeDtypeStruct((B, D), _F32)),
        mesh=mesh,
        scratch_types=(
            pltpu.VMEM_SHARED((V, D2), _F32),
            pltpu.VMEM((nc, _CH), jnp.int32),
            pltpu.VMEM((_CH, D2), _F32),
            pltpu.VMEM((_CH, D2), _F32),
            pltpu.SemaphoreType.DMA,
            pltpu.SemaphoreType.DMA,
            pltpu.SemaphoreType.DMA,
            pltpu.SemaphoreType.DMA,
        ),
    )
    def k(tab_a, idx_a, tab_b, idx_b, out_a, out_b, sp, ixv, r0, r1,
          s0, s1, t0, t1):
        c = lax.axis_index("c")
        sid = lax.axis_index("s")
        wid = c * _NS + sid
        base = wid * pw
        row0 = jnp.minimum(sid * srow, V - srow)
        bufs = (r0, r1)
        gsem = (s0, s1)
        ssem = (t0, t1)

        for tab, idx_hbm, out in ((tab_a, idx_a, out_a),
                                  (tab_b, idx_b, out_b)):
            pltpu.sync_copy(idx_hbm.at[wid], ixv)
            for half in range(2):
                plsc.subcore_barrier()
                pltpu.sync_copy(
                    tab.at[pl.ds(row0, srow), pl.ds(half * D2, D2)],
                    sp.at[pl.ds(row0, srow)])
                plsc.subcore_barrier()

                def gat(j, ping):
                    return pltpu.make_async_copy(sp.at[ixv.at[j]],
                                                 bufs[ping], gsem[ping])

                def sto(j, ping):
                    return pltpu.make_async_copy(
                        bufs[ping],
                        out.at[pl.ds(base + j * _CH, _CH),
                               pl.ds(half * D2, D2)],
                        ssem[ping])

                _gather_pipe(gat, sto, nc)

    return k


def _make_scatter(EP, NP, D):
    pt = EP // _NS
    nc = pt // _CH
    assert nc % 8 == 0
    zr = NP // _NS
    segs = []
    off = 0
    while off < nc:
        n = min(24, nc - off)
        segs.append((off, n))
        off += n
    nch0 = segs[0][1]
    assert all(n % 2 == 0 for _, n in segs)
    mesh = plsc.VectorSubcoreMesh(core_axis_name="c", subcore_axis_name="s")

    @functools.partial(
        pl.kernel,
        out_type=jax.ShapeDtypeStruct((2, NP, D), _F32),
        mesh=mesh,
        scratch_types=(
            pltpu.VMEM_SHARED((NP, D), _F32),
            pltpu.VMEM((2, nch0, _CH), jnp.int32),
            pltpu.VMEM((_CH, D), _F32),
            pltpu.VMEM((_CH, D), _F32),
            pltpu.SemaphoreType.DMA,
            pltpu.SemaphoreType.DMA,
            pltpu.SemaphoreType.DMA,
            pltpu.SemaphoreType.DMA,
        ),
    )
    def k(vals, sidx, zeros, out, acc, ixv, v0, v1, s0, s1, a0, a1):
        c = lax.axis_index("c")
        sid = lax.axis_index("s")
        pltpu.sync_copy(zeros.at[pl.ds(sid * zr, zr)], acc.at[pl.ds(sid * zr, zr)])
        plsc.subcore_barrier()
        base = sid * pt

        def lod(g, buf, sem):
            return pltpu.make_async_copy(
                vals.at[c, pl.ds(base + g * _CH, _CH)], buf, sem)

        def add(i, jloc, buf, sem):
            return pltpu.make_async_copy(buf, acc.at[ixv.at[i, jloc]], sem)

        lod(0, v0, s0).start()
        lod(1, v1, s1).start()

        for j0, nch in segs:
            pltpu.sync_copy(sidx.at[:, sid, pl.ds(j0, nch)],
                            ixv.at[:, pl.ds(0, nch)])

            @pl.loop(0, nch, step=2)
            def _(j):
                g = j0 + j
                lod(g, v0, s0).wait()
                add(0, j, v0, a0).start(add=True)
                add(1, j, v0, a0).start(add=True)
                lod(g + 1, v1, s1).wait()
                add(0, j + 1, v1, a1).start(add=True)
                add(1, j + 1, v1, a1).start(add=True)
                add(0, j, v0, a0).wait()
                add(1, j, v0, a0).wait()

                @pl.when(g + 2 < nc)
                def _():
                    lod(g + 2, v0, s0).start()
                add(0, j + 1, v1, a1).wait()
                add(1, j + 1, v1, a1).wait()

                @pl.when(g + 3 < nc)
                def _():
                    lod(g + 3, v1, s1).start()

        plsc.subcore_barrier()
        pltpu.sync_copy(acc.at[pl.ds(sid * zr, zr)], out.at[c, pl.ds(sid * zr, zr)])

    return k


def _prep_ea(nep, w1ea):
    def body(a_ref, b_ref, o_ref):
        o_ref[...] = jnp.dot(a_ref[...], b_ref[...],
                             preferred_element_type=_F32)
    return pl.pallas_call(
        body,
        out_shape=jax.ShapeDtypeStruct((nep.shape[0], w1ea.shape[1]), _F32),
    )(nep, w1ea)


def _node_phase(aggs_a, aggs_b, na, W2, b2row, W3ab, Nn, H):
    grid = (Nn // _BN,)

    def body(aa_ref, ab_ref, na_ref, w2_ref, b2_ref, w3_ref, ts_ref, td_ref):
        w2 = w2_ref[...]
        hn = jnp.tanh(jnp.dot(aa_ref[0] + ab_ref[0] + na_ref[...], w2,
                              preferred_element_type=_F32) + b2_ref[...])
        dhn = (1.0 - hn * hn) * jnp.dot(aa_ref[1] + ab_ref[1], w2,
                                        preferred_element_type=_F32)
        r = jnp.dot(jnp.concatenate([hn, dhn], axis=0), w3_ref[...],
                    preferred_element_type=_F32)
        ts_ref[...] = jnp.concatenate([r[:_BN, :H], r[_BN:, :H]], axis=1)
        td_ref[...] = jnp.concatenate([r[:_BN, H:], r[_BN:, H:]], axis=1)

    aspec = pl.BlockSpec((2, _BN, H), lambda i: (0, i, 0))
    return pl.pallas_call(
        body,
        grid=grid,
        in_specs=[
            aspec,
            aspec,
            pl.BlockSpec((_BN, H), lambda i: (i, 0)),
            pl.BlockSpec((H, H), lambda i: (0, 0)),
            pl.BlockSpec((1, H), lambda i: (0, 0)),
            pl.BlockSpec((H, 2 * H), lambda i: (0, 0)),
        ],
        out_specs=[pl.BlockSpec((_BN, 2 * H), lambda i: (i, 0))] * 2,
        out_shape=[jax.ShapeDtypeStruct((Nn, 2 * H), _F32)] * 2,
    )(aggs_a, aggs_b, na, W2, b2row, W3ab)


def _edge_phase(first, last, EP, H, dt):
    grid = (EP // _BE,)
    evec = pl.BlockSpec((_BE,), lambda i: (i,))
    emat = pl.BlockSpec((_BE, H), lambda i: (i, 0))
    egat = pl.BlockSpec((_BE, 2 * H), lambda i: (i, 0))
    cons = lambda shp: pl.BlockSpec(shp, lambda i: tuple(0 for _ in shp))
    vspec = pl.BlockSpec((2, _BE, H), lambda i: (0, i, 0))

    def body(*refs):
        if first:
            (x_ref, eps_ref, g_ref, p_ref, vals_ref) = refs
        elif last:
            (x_ref, eps_ref, lp_ref, g_ref, gs_ref, gd_ref, p_ref, w3c_ref,
             out_ref) = refs
        else:
            (x_ref, eps_ref, lp_ref, g_ref, gs_ref, gd_ref, p_ref, w3c_ref,
             xo_ref, lpo_ref, vals_ref) = refs

        p = p_ref[...]
        w1x = p[0][None, :]
        ct0 = p[1][None, :]
        ct1 = p[2][None, :]
        b3r = p[3][None, :]
        w4r = p[4][None, :]
        b4s = p[5, 0]
        xc = x_ref[...][:, None]
        ec = eps_ref[...][:, None]
        g = g_ref[...]
        demdx = ec * w1x

        if not first:
            m = jnp.tanh(xc * w1x + ct0 + g)
            dm = (1.0 - m * m) * demdx
            s2 = jnp.dot(jnp.concatenate([m, dm], axis=0), w3c_ref[...],
                         preferred_element_type=_F32)
            gs = gs_ref[...]
            gd = gd_ref[...]
            s = gs[:, :H] + gd[:, :H] + s2[:_BE] + b3r
            h = jnp.tanh(s)
            ds = gs[:, H:] + gd[:, H:] + s2[_BE:]
            fx = jnp.sum(h * w4r, axis=1, keepdims=True) + b4s
            dout = jnp.sum((1.0 - h * h) * ds * w4r, axis=1, keepdims=True)
            xc = xc + dt * fx
            lp = lp_ref[...][:, None] + dt * (ec * dout)

        if last:
            o = (-0.5 * _LOG2PI) - 0.5 * xc * xc - lp
            out_ref[...] = o[:, 0]
        else:
            if not first:
                xo_ref[...] = xc[:, 0]
                lpo_ref[...] = lp[:, 0]
            m2 = jnp.tanh(xc * w1x + ct1 + g)
            vals_ref[0] = m2
            vals_ref[1] = (1.0 - m2 * m2) * demdx

    if first:
        in_specs = [evec, evec, emat, cons((8, H))]
        out_specs = vspec
        out_shape = jax.ShapeDtypeStruct((2, EP, H), _F32)
    elif last:
        in_specs = [evec, evec, evec, emat, egat, egat, cons((8, H)),
                    cons((H, H))]
        out_specs = evec
        out_shape = jax.ShapeDtypeStruct((EP,), _F32)
    else:
        in_specs = [evec, evec, evec, emat, egat, egat, cons((8, H)),
                    cons((H, H))]
        out_specs = [evec, evec, vspec]
        out_shape = [jax.ShapeDtypeStruct((EP,), _F32),
                     jax.ShapeDtypeStruct((EP,), _F32),
                     jax.ShapeDtypeStruct((2, EP, H), _F32)]

    return pl.pallas_call(body, grid=grid, in_specs=in_specs,
                          out_specs=out_specs, out_shape=out_shape)


def kernel(d, node_type, edge_type, edge_index, node_emb,
           W1, b1, W2, b2, W3, b3, W4, b4, eps):
    E = d.shape[0]
    Nn = node_type.shape[0]
    V, H = node_emb.shape
    n_steps = 2
    dt = 1.0 / n_steps

    EP = _round_up(E, _NW * _CH * 4)
    EPH = EP // 2
    NB = _round_up(Nn, _NW * _CH * 2)
    NP = _round_up(Nn + 1, 128)
    VP = _round_up(V, 8)

    i32 = jnp.int32
    src = edge_index[0].astype(i32)
    dst = edge_index[1].astype(i32)
    epad = EP - E

    def halves(a, fill=0):
        return jnp.pad(a, (0, epad), constant_values=fill).reshape(2, EPH)

    src_g = halves(src).reshape(2, _NW, -1, _CH)
    dst_g = halves(dst).reshape(2, _NW, -1, _CH)
    src_s = halves(src, Nn)
    dst_s = halves(dst, Nn)
    sidx = jnp.stack([src_s, dst_s], axis=1).reshape(2, 2, _NS, -1, _CH)
    et2 = halves(edge_type.astype(i32)).reshape(2, _NW, -1, _CH)
    nt2 = jnp.pad(node_type.astype(i32), (0, NB - Nn)).reshape(_NW, -1, _CH)

    xh = halves(d[:, 0])
    evh = halves(eps[:, 0])
    lph = [jnp.zeros((EPH,), _F32)] * 2
    zerosN = jnp.zeros((NP, H), _F32)

    nep = jnp.pad(node_emb, ((0, VP - V), (0, 0)))
    EA = _prep_ea(nep, W1[2:])
    w1x, w1t = W1[0], W1[1]
    w4r = W4[:, 0]
    b4b = jnp.full((H,), b4[0], _F32)
    zrow = jnp.zeros((H,), _F32)
    b2row = b2.reshape(1, H)
    W3ab = jnp.concatenate([W3[:H], W3[H:2 * H]], axis=1)
    W3c = W3[2 * H:]

    def pv(k0, k1):
        return jnp.stack([w1x, (k0 * dt) * w1t + b1, (k1 * dt) * w1t + b1,
                          b3, w4r, b4b, zrow, zrow])

    gather_prep = _make_gather_small(((VP, EPH), (VP, EPH), (VP, NB)), H)
    scatter = _make_scatter(EPH, NP, H)
    gather_step = _make_gather_big(Nn, 2 * H, EPH)
    edge_first = _edge_phase(True, False, EPH, H, dt)
    edge_mid = _edge_phase(False, False, EPH, H, dt)
    edge_last = _edge_phase(False, True, EPH, H, dt)

    Ga, Gb, NA = gather_prep(EA, et2[0], EA, et2[1], nep, nt2)
    G = (Ga, Gb)
    x = [xh[0], xh[1]]
    lp = lph
    vals = [edge_first(x[hf], evh[hf], G[hf], pv(0, 0)) for hf in range(2)]

    for k in range(n_steps):
        agg_a = scatter(vals[0], sidx[0], zerosN)
        agg_b = scatter(vals[1], sidx[1], zerosN)
        Ts, Td = _node_phase(agg_a, agg_b, NA, W2, b2row, W3ab, Nn, H)
        outs = []
        for hf in range(2):
            Gs, Gd = gather_step(Ts, src_g[hf], Td, dst_g[hf])
            if k < n_steps - 1:
                x[hf], lp[hf], vals[hf] = edge_mid(
                    x[hf], evh[hf], lp[hf], G[hf], Gs, Gd, pv(k, k + 1), W3c)
            else:
                outs.append(edge_last(
                    x[hf], evh[hf], lp[hf], G[hf], Gs, Gd, pv(k, k), W3c))

    return jnp.concatenate(outs)[:E].reshape(E, 1)


# speedup vs baseline: 4.2506x; 1.0142x over previous
import functools
import math

import jax
import jax.numpy as jnp
from jax import lax
from jax.experimental import pallas as pl
from jax.experimental.pallas import tpu as pltpu
from jax.experimental.pallas import tpu_sc as plsc

_NC, _NS = 2, 16
_NW = _NC * _NS
_CH = 128
_BE = 2048
_BN = 2000
_LOG2PI = math.log(2.0 * math.pi)
_F32 = jnp.float32


def _round_up(x, m):
    return (x + m - 1) // m * m


def _gather_pipe(gat, sto, nc):
    gat(0, 0).start()
    gat(1, 1).start()

    @pl.loop(0, nc, step=2)
    def _(j):
        gat(j, 0).wait()
        sto(j, 0).start()
        gat(j + 1, 1).wait()
        sto(j + 1, 1).start()
        sto(j, 0).wait()

        @pl.when(j + 2 < nc)
        def _():
            gat(j + 2, 0).start()
        sto(j + 1, 1).wait()

        @pl.when(j + 3 < nc)
        def _():
            gat(j + 3, 1).start()


def _make_gather_small(specs, D):
    pws = [B // _NW for _, B in specs]
    ncs = [pw // _CH for pw in pws]
    assert all(nc % 2 == 0 for nc in ncs)
    mesh = plsc.VectorSubcoreMesh(core_axis_name="c", subcore_axis_name="s")

    @functools.partial(
        pl.kernel,
        out_type=tuple(jax.ShapeDtypeStruct((B, D), _F32) for _, B in specs),
        mesh=mesh,
        scratch_types=tuple(
            [pltpu.VMEM_SHARED((V, D), _F32) for V, _ in specs]
            + [pltpu.VMEM((nc, _CH), jnp.int32) for nc in ncs]
            + [pltpu.VMEM((_CH, D), _F32), pltpu.VMEM((_CH, D), _F32),
               pltpu.SemaphoreType.DMA, pltpu.SemaphoreType.DMA,
               pltpu.SemaphoreType.DMA, pltpu.SemaphoreType.DMA]),
    )
    def k(*refs):
        n = len(specs)
        tabs = refs[0:2 * n:2]
        idxs = refs[1:2 * n:2]
        outs = refs[2 * n:3 * n]
        sps = refs[3 * n:4 * n]
        ixvs = refs[4 * n:5 * n]
        r0, r1, s0, s1, t0, t1 = refs[5 * n:]
        c = lax.axis_index("c")
        sid = lax.axis_index("s")
        wid = c * _NS + sid

        @pl.when(sid == 0)
        def _():
            for tab, sp in zip(tabs, sps):
                pltpu.sync_copy(tab, sp)
        plsc.subcore_barrier()
        bufs = (r0, r1)
        gsem = (s0, s1)
        ssem = (t0, t1)

        for i in range(n):
            sp, out, ixv, nc, pw = sps[i], outs[i], ixvs[i], ncs[i], pws[i]
            base = wid * pw
            pltpu.sync_copy(idxs[i].at[wid], ixv)

            def gat(j, ping, sp=sp, ixv=ixv):
                return pltpu.make_async_copy(sp.at[ixv.at[j]], bufs[ping],
                                             gsem[ping])

            def sto(j, ping, out=out, base=base):
                return pltpu.make_async_copy(
                    bufs[ping], out.at[pl.ds(base + j * _CH, _CH)], ssem[ping])

            _gather_pipe(gat, sto, nc)

    return k


def _make_gather_big(V, D, B):
    D2 = D // 2
    pw = B // _NW
    nc = pw // _CH
    assert nc % 2 == 0
    srow = _round_up((V + _NS - 1) // _NS, 16)
    mesh = plsc.VectorSubcoreMesh(core_axis_name="c", subcore_axis_name="s")

    @functools.partial(
        pl.kernel,
        out_type=(jax.ShapeDtypeStruct((B, D), _F32),
                  jax.ShapeDtypeStruct((B, D), _F32)),
        mesh=mesh,
        scratch_types=(
            pltpu.VMEM_SHARED((V, D2), _F32),
            pltpu.VMEM((nc, _CH), jnp.int32),
            pltpu.VMEM((_CH, D2), _F32),
            pltpu.VMEM((_CH, D2), _F32),
            pltpu.SemaphoreType.DMA,
            pltpu.SemaphoreType.DMA,
            pltpu.SemaphoreType.DMA,
            pltpu.SemaphoreType.DMA,
        ),
    )
    def k(tab_a, idx_a, tab_b, idx_b, out_a, out_b, sp, ixv, r0, r1,
          s0, s1, t0, t1):
        c = lax.axis_index("c")
        sid = lax.axis_index("s")
        wid = c * _NS + sid
        base = wid * pw
        row0 = jnp.minimum(sid * srow, V - srow)
        bufs = (r0, r1)
        gsem = (s0, s1)
        ssem = (t0, t1)

        for tab, idx_hbm, out in ((tab_a, idx_a, out_a),
                                  (tab_b, idx_b, out_b)):
            pltpu.sync_copy(idx_hbm.at[wid], ixv)
            for half in range(2):
                plsc.subcore_barrier()
                pltpu.sync_copy(
                    tab.at[pl.ds(row0, srow), pl.ds(half * D2, D2)],
                    sp.at[pl.ds(row0, srow)])
                plsc.subcore_barrier()

                def gat(j, ping, ixv=ixv):
                    return pltpu.make_async_copy(sp.at[ixv.at[j]],
                                                 bufs[ping], gsem[ping])

                def sto(j, ping, out=out, half=half):
                    return pltpu.make_async_copy(
                        bufs[ping],
                        out.at[pl.ds(base + j * _CH, _CH),
                               pl.ds(half * D2, D2)],
                        ssem[ping])

                _gather_pipe(gat, sto, nc)

    return k


def _make_scatter(EP, NP, D):
    pt = EP // _NS
    nc = pt // _CH
    assert nc % 8 == 0
    zr = NP // _NS
    segs = []
    off = 0
    while off < nc:
        n = min(24, nc - off)
        segs.append((off, n))
        off += n
    nch0 = segs[0][1]
    assert all(n % 2 == 0 for _, n in segs)
    mesh = plsc.VectorSubcoreMesh(core_axis_name="c", subcore_axis_name="s")

    @functools.partial(
        pl.kernel,
        out_type=jax.ShapeDtypeStruct((2, NP, D), _F32),
        mesh=mesh,
        scratch_types=(
            pltpu.VMEM_SHARED((NP, D), _F32),
            pltpu.VMEM((2, nch0, _CH), jnp.int32),
            pltpu.VMEM((_CH, D), _F32),
            pltpu.VMEM((_CH, D), _F32),
            pltpu.SemaphoreType.DMA,
            pltpu.SemaphoreType.DMA,
            pltpu.SemaphoreType.DMA,
            pltpu.SemaphoreType.DMA,
        ),
    )
    def k(vals, sidx, zeros, out, acc, ixv, v0, v1, s0, s1, a0, a1):
        c = lax.axis_index("c")
        sid = lax.axis_index("s")
        pltpu.sync_copy(zeros.at[pl.ds(sid * zr, zr)], acc.at[pl.ds(sid * zr, zr)])
        plsc.subcore_barrier()
        base = sid * pt

        def lod(g, buf, sem):
            return pltpu.make_async_copy(
                vals.at[c, pl.ds(base + g * _CH, _CH)], buf, sem)

        def add(i, jloc, buf, sem):
            return pltpu.make_async_copy(buf, acc.at[ixv.at[i, jloc]], sem)

        lod(0, v0, s0).start()
        lod(1, v1, s1).start()

        for j0, nch in segs:
            pltpu.sync_copy(sidx.at[:, sid, pl.ds(j0, nch)],
                            ixv.at[:, pl.ds(0, nch)])

            @pl.loop(0, nch, step=2)
            def _(j):
                g = j0 + j
                lod(g, v0, s0).wait()
                add(0, j, v0, a0).start(add=True)
                add(1, j, v0, a0).start(add=True)
                lod(g + 1, v1, s1).wait()
                add(0, j + 1, v1, a1).start(add=True)
                add(1, j + 1, v1, a1).start(add=True)
                add(0, j, v0, a0).wait()
                add(1, j, v0, a0).wait()

                @pl.when(g + 2 < nc)
                def _():
                    lod(g + 2, v0, s0).start()
                add(0, j + 1, v1, a1).wait()
                add(1, j + 1, v1, a1).wait()

                @pl.when(g + 3 < nc)
                def _():
                    lod(g + 3, v1, s1).start()

        plsc.subcore_barrier()
        pltpu.sync_copy(acc.at[pl.ds(sid * zr, zr)], out.at[c, pl.ds(sid * zr, zr)])

    return k


def _prep_ea(nep, w1ea):
    def body(a_ref, b_ref, o_ref):
        o_ref[...] = jnp.dot(a_ref[...], b_ref[...],
                             preferred_element_type=_F32)
    return pl.pallas_call(
        body,
        out_shape=jax.ShapeDtypeStruct((nep.shape[0], w1ea.shape[1]), _F32),
    )(nep, w1ea)


def _node_phase(aggs_a, aggs_b, na, W2, b2row, W3ab, Nn, H):
    grid = (Nn // _BN,)

    def body(aa_ref, ab_ref, na_ref, w2_ref, b2_ref, w3_ref, ts_ref, td_ref):
        w2 = w2_ref[...]
        hn = jnp.tanh(jnp.dot(aa_ref[0] + ab_ref[0] + na_ref[...], w2,
                              preferred_element_type=_F32) + b2_ref[...])
        dhn = (1.0 - hn * hn) * jnp.dot(aa_ref[1] + ab_ref[1], w2,
                                        preferred_element_type=_F32)
        r = jnp.dot(jnp.concatenate([hn, dhn], axis=0), w3_ref[...],
                    preferred_element_type=_F32)
        ts_ref[...] = jnp.concatenate([r[:_BN, :H], r[_BN:, :H]], axis=1)
        td_ref[...] = jnp.concatenate([r[:_BN, H:], r[_BN:, H:]], axis=1)

    aspec = pl.BlockSpec((2, _BN, H), lambda i: (0, i, 0))
    return pl.pallas_call(
        body,
        grid=grid,
        in_specs=[
            aspec,
            aspec,
            pl.BlockSpec((_BN, H), lambda i: (i, 0)),
            pl.BlockSpec((H, H), lambda i: (0, 0)),
            pl.BlockSpec((1, H), lambda i: (0, 0)),
            pl.BlockSpec((H, 2 * H), lambda i: (0, 0)),
        ],
        out_specs=[pl.BlockSpec((_BN, 2 * H), lambda i: (i, 0))] * 2,
        out_shape=[jax.ShapeDtypeStruct((Nn, 2 * H), _F32)] * 2,
    )(aggs_a, aggs_b, na, W2, b2row, W3ab)


def _edge_phase(first, last, EP, H, dt):
    grid = (EP // _BE,)
    evec = pl.BlockSpec((_BE,), lambda i: (i,))
    emat = pl.BlockSpec((_BE, H), lambda i: (i, 0))
    egat = pl.BlockSpec((_BE, 2 * H), lambda i: (i, 0))
    cons = lambda shp: pl.BlockSpec(shp, lambda i: tuple(0 for _ in shp))
    vspec = pl.BlockSpec((2, _BE, H), lambda i: (0, i, 0))

    def body(*refs):
        if first:
            (x_ref, eps_ref, g_ref, p_ref, vals_ref) = refs
        elif last:
            (x_ref, eps_ref, lp_ref, g_ref, gs_ref, gd_ref, p_ref, w3c_ref,
             out_ref) = refs
        else:
            (x_ref, eps_ref, lp_ref, g_ref, gs_ref, gd_ref, p_ref, w3c_ref,
             xo_ref, lpo_ref, vals_ref) = refs

        p = p_ref[...]
        w1x = p[0][None, :]
        ct0 = p[1][None, :]
        ct1 = p[2][None, :]
        b3r = p[3][None, :]
        w4r = p[4][None, :]
        b4s = p[5, 0]
        xc = x_ref[...][:, None]
        ec = eps_ref[...][:, None]
        g = g_ref[...]
        demdx = ec * w1x

        if not first:
            m = jnp.tanh(xc * w1x + ct0 + g)
            dm = (1.0 - m * m) * demdx
            s2 = jnp.dot(jnp.concatenate([m, dm], axis=0), w3c_ref[...],
                         preferred_element_type=_F32)
            gs = gs_ref[...]
            gd = gd_ref[...]
            s = gs[:, :H] + gd[:, :H] + s2[:_BE] + b3r
            h = jnp.tanh(s)
            ds = gs[:, H:] + gd[:, H:] + s2[_BE:]
            fx = jnp.sum(h * w4r, axis=1, keepdims=True) + b4s
            dout = jnp.sum((1.0 - h * h) * ds * w4r, axis=1, keepdims=True)
            xc = xc + dt * fx
            lp = lp_ref[...][:, None] + dt * (ec * dout)

        if last:
            o = (-0.5 * _LOG2PI) - 0.5 * xc * xc - lp
            out_ref[...] = o[:, 0]
        else:
            if not first:
                xo_ref[...] = xc[:, 0]
                lpo_ref[...] = lp[:, 0]
            m2 = jnp.tanh(xc * w1x + ct1 + g)
            vals_ref[0] = m2
            vals_ref[1] = (1.0 - m2 * m2) * demdx

    gspecs = [egat, egat]
    if first:
        in_specs = [evec, evec, emat, cons((8, H))]
        out_specs = vspec
        out_shape = jax.ShapeDtypeStruct((2, EP, H), _F32)
    elif last:
        in_specs = [evec, evec, evec, emat] + gspecs + [cons((8, H)),
                                                        cons((H, H))]
        out_specs = evec
        out_shape = jax.ShapeDtypeStruct((EP,), _F32)
    else:
        in_specs = [evec, evec, evec, emat] + gspecs + [cons((8, H)),
                                                        cons((H, H))]
        out_specs = [evec, evec, vspec]
        out_shape = [jax.ShapeDtypeStruct((EP,), _F32),
                     jax.ShapeDtypeStruct((EP,), _F32),
                     jax.ShapeDtypeStruct((2, EP, H), _F32)]

    return pl.pallas_call(body, grid=grid, in_specs=in_specs,
                          out_specs=out_specs, out_shape=out_shape)


def kernel(d, node_type, edge_type, edge_index, node_emb,
           W1, b1, W2, b2, W3, b3, W4, b4, eps):
    E = d.shape[0]
    Nn = node_type.shape[0]
    V, H = node_emb.shape
    n_steps = 2
    dt = 1.0 / n_steps

    EP = _round_up(E, _NW * _CH * 4)
    EPH = EP // 2
    NB = _round_up(Nn, _NW * _CH * 2)
    NP = _round_up(Nn + 1, 128)
    VP = _round_up(V, 8)

    i32 = jnp.int32
    src = edge_index[0].astype(i32)
    dst = edge_index[1].astype(i32)
    epad = EP - E

    def halves(a, fill=0):
        return jnp.pad(a, (0, epad), constant_values=fill).reshape(2, EPH)

    src_g = halves(src).reshape(2, _NW, -1, _CH)
    dst_g = halves(dst).reshape(2, _NW, -1, _CH)
    src_s = halves(src, Nn)
    dst_s = halves(dst, Nn)
    sidx = jnp.stack([src_s, dst_s], axis=1).reshape(2, 2, _NS, -1, _CH)
    et2 = halves(edge_type.astype(i32)).reshape(2, _NW, -1, _CH)
    nt2 = jnp.pad(node_type.astype(i32), (0, NB - Nn)).reshape(_NW, -1, _CH)

    xh = halves(d[:, 0])
    evh = halves(eps[:, 0])
    lph = [jnp.zeros((EPH,), _F32)] * 2
    zerosN = jnp.zeros((NP, H), _F32)

    nep = jnp.pad(node_emb, ((0, VP - V), (0, 0)))
    EA = _prep_ea(nep, W1[2:])
    w1x, w1t = W1[0], W1[1]
    w4r = W4[:, 0]
    b4b = jnp.full((H,), b4[0], _F32)
    zrow = jnp.zeros((H,), _F32)
    b2row = b2.reshape(1, H)
    W3ab = jnp.concatenate([W3[:H], W3[H:2 * H]], axis=1)
    W3c = W3[2 * H:]

    def pv(k0, k1):
        return jnp.stack([w1x, (k0 * dt) * w1t + b1, (k1 * dt) * w1t + b1,
                          b3, w4r, b4b, zrow, zrow])

    gather_prep_a = _make_gather_small(((VP, EPH),), H)
    gather_prep_b = _make_gather_small(((VP, EPH), (VP, NB)), H)
    scatter = _make_scatter(EPH, NP, H)
    gather_step = _make_gather_big(Nn, 2 * H, EPH)
    edge_first = _edge_phase(True, False, EPH, H, dt)
    edge_mid = _edge_phase(False, False, EPH, H, dt)
    edge_last = _edge_phase(False, True, EPH, H, dt)

    Ga, = gather_prep_a(EA, et2[0])
    Gb, NA = gather_prep_b(EA, et2[1], nep, nt2)
    G = (Ga, Gb)
    x = [xh[0], xh[1]]
    lp = lph
    vals = [edge_first(x[hf], evh[hf], G[hf], pv(0, 0)) for hf in range(2)]

    for k in range(n_steps):
        agg_a = scatter(vals[0], sidx[0], zerosN)
        agg_b = scatter(vals[1], sidx[1], zerosN)
        Ts, Td = _node_phase(agg_a, agg_b, NA, W2, b2row, W3ab, Nn, H)
        outs = []
        for hf in range(2):
            Gs, Gd = gather_step(Ts, src_g[hf], Td, dst_g[hf])
            if k < n_steps - 1:
                x[hf], lp[hf], vals[hf] = edge_mid(
                    x[hf], evh[hf], lp[hf], G[hf], Gs, Gd, pv(k, k + 1), W3c)
            else:
                outs.append(edge_last(
                    x[hf], evh[hf], lp[hf], G[hf], Gs, Gd, pv(k, k), W3c))

    return jnp.concatenate(outs)[:E].reshape(E, 1)
